# baseline (speedup 1.0000x reference)
import jax
import jax.numpy as jnp
from jax import lax
from jax.experimental import pallas as pl
from jax.experimental.pallas import tpu as pltpu

B, T, S, V = 8, 32, 200, 100000
NROWS = B * T
R = 16
SP = 256


def _tc_body(x_ref, w_ref, att_ref, src_ref, src_sm, b_ref, y_ref, g_sc, val_sc):
    bidx = pl.program_id(0) // (T // R)
    x = x_ref[...]
    w = w_ref[...]
    m = jnp.max(x, axis=1, keepdims=True)
    sumexp = jnp.sum(jnp.exp(x - m), axis=1, keepdims=True)
    dot = jnp.sum(x * w, axis=1, keepdims=True)
    p = jax.nn.sigmoid(dot + b_ref[0, 0])
    scale = (1.0 - p) / sumexp
    q = m - jnp.log(scale)
    y_ref[...] = jnp.log(jnp.exp(x - q) + 1e-10)

    lane128 = lax.broadcasted_iota(jnp.int32, (1, 128), 1)
    for s in range(S):
        col = src_sm[bidx, s]
        hi = pl.multiple_of((col // 128) * 128, 128)
        xs = x_ref[:, pl.ds(hi, 128)]
        sel = lane128 == (col - hi)
        g_sc[:, s : s + 1] = jnp.sum(
            jnp.where(sel, xs, 0.0), axis=1, keepdims=True
        )

    lane = lax.broadcasted_iota(jnp.int32, (1, SP), 1)
    att = jnp.where(lane < S, att_ref[...], 0.0)
    copy = p * att
    src = jnp.where(lane < S, src_ref[0], -1)
    eq = (src[0, :, None] == src[0, None, :]).astype(jnp.float32)
    tot = jnp.dot(copy, eq, preferred_element_type=jnp.float32)

    g = g_sc[...]
    val_sc[...] = jnp.log(jnp.exp(g - q) + 1e-10 + tot)

    for s in range(S):
        col = src_sm[bidx, s]
        hi = pl.multiple_of((col // 128) * 128, 128)
        ys = y_ref[:, pl.ds(hi, 128)]
        sel = lane128 == (col - hi)
        y_ref[:, pl.ds(hi, 128)] = jnp.where(sel, val_sc[:, s : s + 1], ys)


def _tc_fused(x2, w2, att2, src2, b2):
    blocks_per_batch = T // R
    return pl.pallas_call(
        _tc_body,
        grid=(NROWS // R,),
        in_specs=[
            pl.BlockSpec((R, V), lambda i: (i, 0)),
            pl.BlockSpec((1, V), lambda i: (0, 0)),
            pl.BlockSpec((R, SP), lambda i: (i, 0)),
            pl.BlockSpec((1, 1, SP), lambda i: (i // blocks_per_batch, 0, 0)),
            pl.BlockSpec(memory_space=pltpu.SMEM),
            pl.BlockSpec((1, 1), lambda i: (0, 0)),
        ],
        out_specs=pl.BlockSpec((R, V), lambda i: (i, 0)),
        out_shape=jax.ShapeDtypeStruct((NROWS, V), jnp.float32),
        scratch_shapes=[
            pltpu.VMEM((R, SP), jnp.float32),
            pltpu.VMEM((R, SP), jnp.float32),
        ],
    )(x2, w2, att2, src2.reshape(B, 1, SP), src2, b2)


def kernel(src, output, attention, W, b):
    src = src.astype(jnp.int32)
    x2 = output.reshape(NROWS, V)
    w2 = W.reshape(1, V)
    b2 = b.reshape(1, 1)
    att2 = jnp.pad(attention.reshape(NROWS, S), ((0, 0), (0, SP - S)))
    src2 = jnp.pad(src, ((0, 0), (0, SP - S)))
    y = _tc_fused(x2, w2, att2, src2, b2)
    return (y.reshape(B, T, V), attention)


# speedup vs baseline: 1.1263x; 1.1263x over previous
import jax
import jax.numpy as jnp
from jax import lax
from jax.experimental import pallas as pl
from jax.experimental.pallas import tpu as pltpu

B, T, S, V = 8, 32, 200, 100000
NROWS = B * T
R = 16
SP = 256


def _tc_body(x_ref, w_ref, att_ref, src_ref, src_sm, b_ref, y_ref, g_sc, val_sc):
    bidx = pl.program_id(0) // (T // R)
    x = x_ref[...]
    w = w_ref[...]
    m = jnp.max(x, axis=1, keepdims=True)
    e = jnp.exp(x - m)
    sumexp = jnp.sum(e, axis=1, keepdims=True)
    dot = jnp.sum(x * w, axis=1, keepdims=True)
    p = jax.nn.sigmoid(dot + b_ref[0, 0])
    scale = (1.0 - p) / sumexp
    y_ref[...] = jnp.log(e * scale + 1e-10)

    lane128 = lax.broadcasted_iota(jnp.int32, (1, 128), 1)
    for s in range(S):
        col = src_sm[bidx, s]
        hi = pl.multiple_of((col // 128) * 128, 128)
        xs = x_ref[:, pl.ds(hi, 128)]
        sel = lane128 == (col - hi)
        g_sc[:, s : s + 1] = jnp.sum(
            jnp.where(sel, xs, 0.0), axis=1, keepdims=True
        )

    lane = lax.broadcasted_iota(jnp.int32, (1, SP), 1)
    att = jnp.where(lane < S, att_ref[...], 0.0)
    copy = p * att
    src = jnp.where(lane < S, src_ref[0], -1)
    eq = (src[0, :, None] == src[0, None, :]).astype(jnp.float32)
    tot = jnp.dot(copy, eq, preferred_element_type=jnp.float32)

    g = g_sc[...]
    val_sc[...] = jnp.log(jnp.exp(g - m) * scale + 1e-10 + tot)

    for s in range(S):
        col = src_sm[bidx, s]
        hi = pl.multiple_of((col // 128) * 128, 128)
        ys = y_ref[:, pl.ds(hi, 128)]
        sel = lane128 == (col - hi)
        y_ref[:, pl.ds(hi, 128)] = jnp.where(sel, val_sc[:, s : s + 1], ys)


def _tc_fused(x2, w2, att2, src2, b2):
    blocks_per_batch = T // R
    return pl.pallas_call(
        _tc_body,
        grid=(NROWS // R,),
        in_specs=[
            pl.BlockSpec((R, V), lambda i: (i, 0)),
            pl.BlockSpec((1, V), lambda i: (0, 0)),
            pl.BlockSpec((R, SP), lambda i: (i, 0)),
            pl.BlockSpec((1, 1, SP), lambda i: (i // blocks_per_batch, 0, 0)),
            pl.BlockSpec(memory_space=pltpu.SMEM),
            pl.BlockSpec((1, 1), lambda i: (0, 0)),
        ],
        out_specs=pl.BlockSpec((R, V), lambda i: (i, 0)),
        out_shape=jax.ShapeDtypeStruct((NROWS, V), jnp.float32),
        scratch_shapes=[
            pltpu.VMEM((R, SP), jnp.float32),
            pltpu.VMEM((R, SP), jnp.float32),
        ],
    )(x2, w2, att2, src2.reshape(B, 1, SP), src2, b2)


def kernel(src, output, attention, W, b):
    src = src.astype(jnp.int32)
    x2 = output.reshape(NROWS, V)
    w2 = W.reshape(1, V)
    b2 = b.reshape(1, 1)
    att2 = jnp.pad(attention.reshape(NROWS, S), ((0, 0), (0, SP - S)))
    src2 = jnp.pad(src, ((0, 0), (0, SP - S)))
    y = _tc_fused(x2, w2, att2, src2, b2)
    return (y.reshape(B, T, V), attention)
